# Initial kernel scaffold; baseline (speedup 1.0000x reference)
import jax
import jax.numpy as jnp
from jax.experimental import pallas as pl


def kernel(x, ptrs, csr):
    raise NotImplementedError("write your pallas kernel here")


# speedup vs baseline: 135.8622x; 135.8622x over previous
import functools

import jax
import jax.numpy as jnp
from jax import lax
from jax.experimental import pallas as pl
from jax.experimental.pallas import tpu as pltpu
from jax.experimental.pallas import tpu_sc as plsc

_N_SEG = 100000


def _make_sc_kernel(n_seg_pad, edges_per_worker, chunk, num_cores, num_subcores):
    part = n_seg_pad // num_subcores
    mesh = plsc.VectorSubcoreMesh(core_axis_name="c", subcore_axis_name="s")

    @functools.partial(
        pl.kernel,
        out_type=jax.ShapeDtypeStruct((num_cores * n_seg_pad,), jnp.float32),
        mesh=mesh,
        scratch_types=[
            pltpu.VMEM((chunk,), jnp.int32),
            pltpu.VMEM((chunk,), jnp.int32),
            pltpu.VMEM((chunk,), jnp.float32),
            pltpu.VMEM((part,), jnp.float32),
            pltpu.VMEM_SHARED((n_seg_pad,), jnp.float32),
            pltpu.SemaphoreType.DMA,
        ],
    )
    def run(x_hbm, ptrs_hbm, csr_hbm, out_hbm,
            ptrs_v, csr_v, vals_v, bounce_v, acc, sem):
        c = lax.axis_index("c")
        s = lax.axis_index("s")
        wid = s * num_cores + c

        zvec = jnp.zeros((16,), jnp.float32)

        def zbody(i, carry):
            bounce_v[pl.ds(i * 16, 16)] = zvec
            return carry

        lax.fori_loop(0, part // 16, zbody, 0)
        pltpu.sync_copy(bounce_v, acc.at[pl.ds(s * part, part)])
        plsc.subcore_barrier()

        nchunks = edges_per_worker // chunk

        def body(i, carry):
            base = wid * edges_per_worker + i * chunk
            pltpu.sync_copy(ptrs_hbm.at[pl.ds(base, chunk)], ptrs_v)
            pltpu.async_copy(x_hbm.at[ptrs_v], vals_v, sem).wait()
            pltpu.sync_copy(csr_hbm.at[pl.ds(base, chunk)], csr_v)
            pltpu.sync_copy(vals_v, acc.at[csr_v], add=True)
            return carry

        lax.fori_loop(0, nchunks, body, 0)

        plsc.subcore_barrier()
        pltpu.sync_copy(acc.at[pl.ds(s * part, part)], bounce_v)
        pltpu.sync_copy(bounce_v,
                        out_hbm.at[pl.ds(c * n_seg_pad + s * part, part)])

    return run


def _make_combine(n_seg_pad):
    def _combine_body(p_ref, o_ref):
        o_ref[...] = (p_ref[pl.ds(0, n_seg_pad)]
                      + p_ref[pl.ds(n_seg_pad, n_seg_pad)])
    return _combine_body


def kernel(x, ptrs, csr):
    n_edges = ptrs.shape[0]
    info = plsc.get_sparse_core_info()
    num_cores, num_subcores = info.num_cores, info.num_subcores
    n_workers = num_cores * num_subcores
    assert n_edges % n_workers == 0
    edges_per_worker = n_edges // n_workers
    chunk = 10000
    assert edges_per_worker % chunk == 0

    align = num_subcores * 8
    n_seg_pad = ((_N_SEG + align - 1) // align) * align

    run = _make_sc_kernel(n_seg_pad, edges_per_worker, chunk,
                          num_cores, num_subcores)
    partials = run(x, ptrs, csr)

    combined = pl.pallas_call(
        _make_combine(n_seg_pad),
        out_shape=jax.ShapeDtypeStruct((n_seg_pad,), jnp.float32),
    )(partials)
    return combined[:_N_SEG]


# speedup vs baseline: 254.9406x; 1.8765x over previous
import functools

import jax
import jax.numpy as jnp
from jax import lax
from jax.experimental import pallas as pl
from jax.experimental.pallas import tpu as pltpu
from jax.experimental.pallas import tpu_sc as plsc

_N_SEG = 100000


def _make_sc_kernel(n_seg_pad, n_nodes_pad, edges_per_worker, chunk,
                    num_cores, num_subcores):
    part = n_seg_pad // num_subcores
    xpart = n_nodes_pad // num_subcores
    mesh = plsc.VectorSubcoreMesh(core_axis_name="c", subcore_axis_name="s")
    nchunks = edges_per_worker // chunk
    assert nchunks % 2 == 0 and nchunks >= 4

    @functools.partial(
        pl.kernel,
        out_type=jax.ShapeDtypeStruct((num_cores * n_seg_pad,), jnp.float32),
        mesh=mesh,
        scratch_types=[
            pltpu.VMEM((chunk,), jnp.int32),
            pltpu.VMEM((chunk,), jnp.int32),
            pltpu.VMEM((chunk,), jnp.int32),
            pltpu.VMEM((chunk,), jnp.int32),
            pltpu.VMEM((chunk,), jnp.float32),
            pltpu.VMEM((chunk,), jnp.float32),
            pltpu.VMEM((part,), jnp.float32),
            pltpu.VMEM_SHARED((n_nodes_pad,), jnp.float32),
            pltpu.VMEM_SHARED((n_seg_pad,), jnp.float32),
            pltpu.SemaphoreType.DMA((2,)),
            pltpu.SemaphoreType.DMA((2,)),
        ],
    )
    def run(x_hbm, ptrs_hbm, csr_hbm, out_hbm,
            ptrs_v0, ptrs_v1, csr_v0, csr_v1, vals_v0, vals_v1,
            bounce_v, x_spm, acc, gsem, ssem):
        ptrs_v = (ptrs_v0, ptrs_v1)
        csr_v = (csr_v0, csr_v1)
        vals_v = (vals_v0, vals_v1)
        c = lax.axis_index("c")
        s = lax.axis_index("s")
        wid = s * num_cores + c
        ebase = wid * edges_per_worker

        pltpu.sync_copy(x_hbm.at[pl.ds(s * xpart, xpart)],
                        bounce_v.at[pl.ds(0, xpart)])
        pltpu.sync_copy(bounce_v.at[pl.ds(0, xpart)],
                        x_spm.at[pl.ds(s * xpart, xpart)])

        zvec = jnp.zeros((16,), jnp.float32)

        def zbody(i, carry):
            bounce_v[pl.ds(i * 16, 16)] = zvec
            return carry

        lax.fori_loop(0, part // 16, zbody, 0)
        pltpu.sync_copy(bounce_v, acc.at[pl.ds(s * part, part)])
        plsc.subcore_barrier()

        def load(i, b):
            base = ebase + i * chunk
            pltpu.sync_copy(ptrs_hbm.at[pl.ds(base, chunk)], ptrs_v[b])
            pltpu.sync_copy(csr_hbm.at[pl.ds(base, chunk)], csr_v[b])

        def gather(b):
            pltpu.async_copy(x_spm.at[ptrs_v[b]], vals_v[b],
                             gsem.at[b]).wait()

        def scat_start(b):
            pltpu.async_copy(vals_v[b], acc.at[csr_v[b]],
                             ssem.at[b], add=True)

        def scat_wait(b):
            pltpu.make_async_copy(vals_v[b], acc.at[csr_v[b]],
                                  ssem.at[b]).wait()

        for b in range(2):
            load(b, b)
            gather(b)
            scat_start(b)

        @pl.loop(2, nchunks, step=2)
        def pair(io):
            for b in range(2):
                scat_wait(b)
                load(io + b, b)
                gather(b)
                scat_start(b)

        scat_wait(0)
        scat_wait(1)

        plsc.subcore_barrier()
        pltpu.sync_copy(acc.at[pl.ds(s * part, part)], bounce_v)
        pltpu.sync_copy(bounce_v,
                        out_hbm.at[pl.ds(c * n_seg_pad + s * part, part)])

    return run


def _make_combine(n_seg_pad):
    def _combine_body(p_ref, o_ref):
        o_ref[...] = (p_ref[pl.ds(0, n_seg_pad)]
                      + p_ref[pl.ds(n_seg_pad, n_seg_pad)])
    return _combine_body


def kernel(x, ptrs, csr):
    n_edges = ptrs.shape[0]
    n_nodes = x.shape[0]
    info = plsc.get_sparse_core_info()
    num_cores, num_subcores = info.num_cores, info.num_subcores
    n_workers = num_cores * num_subcores
    assert n_edges % n_workers == 0
    edges_per_worker = n_edges // n_workers
    chunk = 10000
    assert edges_per_worker % chunk == 0

    align = num_subcores * 8
    n_seg_pad = ((_N_SEG + align - 1) // align) * align
    n_nodes_pad = ((n_nodes + align - 1) // align) * align

    x_pad = jnp.zeros((n_nodes_pad,), jnp.float32).at[:n_nodes].set(x)
    run = _make_sc_kernel(n_seg_pad, n_nodes_pad, edges_per_worker, chunk,
                          num_cores, num_subcores)
    partials = run(x_pad, ptrs, csr)

    combined = pl.pallas_call(
        _make_combine(n_seg_pad),
        out_shape=jax.ShapeDtypeStruct((n_seg_pad,), jnp.float32),
    )(partials)
    return combined[:_N_SEG]
